# Initial kernel scaffold; baseline (speedup 1.0000x reference)
import jax
import jax.numpy as jnp
from jax.experimental import pallas as pl


def kernel(logits, temperature, token_lengths, top_k, top_p):
    raise NotImplementedError("write your pallas kernel here")


# speedup vs baseline: 41.2769x; 41.2769x over previous
import functools

import jax
import jax.numpy as jnp
from jax import lax
from jax.experimental import pallas as pl
from jax.experimental.pallas import tpu as pltpu
from jax.experimental.pallas import tpu_sc as plsc

_B = 128
_V = 100000
_K = 50
_KP = 64
_L = 16
_NVR = _V // _L
_CAP = 2048
_NW = 32
_RW = _B // _NW
_NEG = -3.0e38
_BIG = 3.0e38
_FIN = -1.0e37
_EPS = 1e-5
_EOS = 2
_MINP = 0.001
_EOSR = 100.0
_LNEG = -1.0e30


def _sc_body(logits_hbm, temp_hbm, tl_hbm, topk_hbm, topp_hbm, out_hbm,
             row_v, cand_v, cand_i, top_v, top_i, temp_v, topk_v, topp_v,
             mark_v, pad_v, fidx_v, lens_v, outb_v, sem):
  cid = lax.axis_index("c")
  sid = lax.axis_index("s")
  wid = sid * 2 + cid

  pltpu.sync_copy(temp_hbm, temp_v)
  pltpu.sync_copy(topk_hbm, topk_v)
  pltpu.sync_copy(topp_hbm, topp_v)

  lanes = lax.iota(jnp.int32, _L)
  zeros_i = lanes * 0
  ones_i = zeros_i + 1
  neg16 = jnp.full((_L,), _NEG, jnp.float32)
  lane0 = lanes == 0

  def store1(ref, pos, val):
    plsc.store_scatter(ref, [jnp.broadcast_to(pos, (_L,))],
                       jnp.broadcast_to(val, (_L,)), mask=lane0)

  def load1(ref, pos):
    return plsc.load_gather(ref, [jnp.broadcast_to(pos, (_L,))])[0]

  def row_body(r, _carry):
    b = wid * _RW + r
    pltpu.sync_copy(logits_hbm.at[b], row_v)

    def p1(j, c):
      r0, r1, r2, r3 = c
      x = row_v[pl.ds(pl.multiple_of(j * _L, _L), _L)]
      m0 = jnp.maximum(r0, x)
      y = jnp.minimum(r0, x)
      m1 = jnp.maximum(r1, y)
      y = jnp.minimum(r1, y)
      m2 = jnp.maximum(r2, y)
      y = jnp.minimum(r2, y)
      m3 = jnp.maximum(r3, y)
      return (m0, m1, m2, m3)

    c0, c1, c2, c3 = lax.fori_loop(0, _NVR, p1, (neg16, neg16, neg16, neg16),
                                   unroll=4)

    def pt(_i, c):
      a0, a1, a2, a3 = c
      s = jnp.max(jnp.maximum(jnp.maximum(a0, a1), jnp.maximum(a2, a3)))
      a0 = jnp.where(a0 == s, _NEG, a0)
      a1 = jnp.where(a1 == s, _NEG, a1)
      a2 = jnp.where(a2 == s, _NEG, a2)
      a3 = jnp.where(a3 == s, _NEG, a3)
      return (a0, a1, a2, a3)

    a0, a1, a2, a3 = lax.fori_loop(0, _K - 1, pt, (c0, c1, c2, c3))
    t = jnp.max(jnp.maximum(jnp.maximum(a0, a1), jnp.maximum(a2, a3)))

    def p2(j, off):
      base = pl.multiple_of(j * _L, _L)
      x = row_v[pl.ds(base, _L)]
      msk = x >= t
      plsc.store_compressed(cand_v.at[pl.ds(off, _L)], x, mask=msk)
      plsc.store_compressed(cand_i.at[pl.ds(off, _L)], lanes + j * _L,
                            mask=msk)
      cnt = jnp.max(plsc.all_reduce_population_count(msk))
      return jnp.minimum(off + cnt, _CAP - _L)

    off = lax.fori_loop(0, _NVR, p2, jnp.int32(0), unroll=4)
    cand_v[pl.ds(off, _L)] = neg16
    nv = (off + _L - 1) >> 4

    for l in range(4):
      top_v[pl.ds(l * _L, _L)] = neg16
      top_i[pl.ds(l * _L, _L)] = zeros_i

    def ext(q, _c):
      def fa(j, cur):
        v = cand_v[pl.ds(pl.multiple_of(j * _L, _L), _L)]
        return jnp.maximum(cur, jnp.max(v))

      s = lax.fori_loop(0, nv, fa, jnp.float32(_NEG))

      def fb(j, pos):
        v = cand_v[pl.ds(pl.multiple_of(j * _L, _L), _L)]
        f = jnp.max(plsc.all_reduce_ffs(v == s))
        return jnp.where((pos < 0) & (f < _L), j * _L + f, pos)

      pos = lax.fori_loop(0, nv, fb, jnp.int32(-1))
      store1(top_v, q, s)
      store1(top_i, q, load1(cand_i, pos))
      store1(cand_v, pos, jnp.float32(_NEG))
      return 0

    lax.fori_loop(0, _K, ext, 0)

    tmp = load1(temp_v, b)
    kk = jnp.clip(load1(topk_v, b), 1, _K)
    q1p = jnp.float32(1.0) - load1(topp_v, b)

    v = [top_v[pl.ds(l * _L, _L)] for l in range(4)]
    ii = [top_i[pl.ds(l * _L, _L)] for l in range(4)]
    rank = [lanes + l * _L for l in range(4)]

    tmp_v = jnp.broadcast_to(tmp, (_L,))
    w = [jnp.where(vl > _FIN, vl / tmp_v, _NEG) for vl in v]

    mm = [jnp.where(rank[l] < kk, w[l], _BIG) for l in range(4)]
    thr = jnp.min(jnp.minimum(jnp.minimum(mm[0], mm[1]),
                              jnp.minimum(mm[2], mm[3])))
    wk = [jnp.where(w[l] >= thr, w[l], _NEG) for l in range(4)]

    mx = jnp.max(jnp.maximum(jnp.maximum(wk[0], wk[1]),
                             jnp.maximum(wk[2], wk[3])))
    e = [jnp.exp(wk[l] - mx) for l in range(4)]
    tot = jnp.broadcast_to(jnp.sum(((e[0] + e[1]) + (e[2] + e[3]))), (_L,))
    d = [e[l] / tot for l in range(4)]

    asc = [jnp.flip(d[3 - l], axis=0) for l in range(4)]
    base = jnp.float32(0.0)
    acum = []
    for l in range(4):
      acum.append(jnp.cumsum(asc[l]) + base)
      base = base + jnp.sum(asc[l])
    s_desc = [jnp.flip(acum[3 - l], axis=0) for l in range(4)]

    cond = [s_desc[l] <= q1p for l in range(4)]
    cond[0] = cond[0] & (lanes > 0)
    w2 = [jnp.where(cond[l], _NEG, wk[l]) for l in range(4)]

    mx2 = jnp.max(jnp.maximum(jnp.maximum(w2[0], w2[1]),
                              jnp.maximum(w2[2], w2[3])))
    e2 = [jnp.exp(w2[l] - mx2) for l in range(4)]
    tot2 = jnp.broadcast_to(jnp.sum(((e2[0] + e2[1]) + (e2[2] + e2[3]))),
                            (_L,))
    pr = [e2[l] / tot2 for l in range(4)]

    sv = [w2[l] > _FIN for l in range(4)]
    nsur = jnp.int32(0)
    for l in range(4):
      nsur = nsur + jnp.max(plsc.all_reduce_population_count(sv[l]))
    mpad = _K - nsur

    for l2 in range(8):
      mark_v[pl.ds(l2 * _L, _L)] = zeros_i
    for l in range(4):
      plsc.store_scatter(mark_v, [ii[l]], ones_i,
                         mask=sv[l] & (ii[l] < 128))

    def pp_body(l2, fc):
      mk = mark_v[pl.ds(pl.multiple_of(l2 * _L, _L), _L)]
      free = mk == 0
      fcum = jnp.cumsum(jnp.where(free, 1, 0)) + fc
      sel = free & (fcum <= mpad)
      woff = jnp.minimum(fc, mpad)
      plsc.store_compressed(pad_v.at[pl.ds(woff, _L)], lanes + l2 * _L,
                            mask=sel)
      return fc + jnp.max(plsc.all_reduce_population_count(free))

    lax.fori_loop(0, 8, pp_body, jnp.int32(0))

    hcarry = jnp.int32(0)
    fi = []
    for l in range(4):
      hole = (rank[l] < _K) & jnp.logical_not(sv[l])
      hc = jnp.cumsum(jnp.where(hole, 1, 0)) + hcarry
      g = plsc.load_gather(pad_v, [hc - 1], mask=hole)
      f = jnp.where(hole, g, ii[l])
      f = jnp.where(rank[l] < _K, f, 0)
      f = jnp.clip(f, 0, _V - 1)
      fi.append(f)
      fidx_v[pl.ds(l * _L, _L)] = f
      hcarry = hcarry + jnp.max(plsc.all_reduce_population_count(hole))

    pltpu.async_copy(tl_hbm.at[fidx_v], lens_v, sem).wait()

    lw = [jnp.where(rank[l] < _K,
                    lens_v[pl.ds(l * _L, _L)].astype(jnp.float32), _LNEG)
          for l in range(4)]
    lmx = jnp.max(jnp.maximum(jnp.maximum(lw[0], lw[1]),
                              jnp.maximum(lw[2], lw[3])))
    le = [jnp.exp(lw[l] - lmx) for l in range(4)]
    ltot = jnp.broadcast_to(jnp.sum(((le[0] + le[1]) + (le[2] + le[3]))),
                            (_L,))
    ls = [le[l] / ltot for l in range(4)]

    mix = [jnp.float32(0.5) * pr[l] + jnp.float32(0.5) * ls[l]
           for l in range(4)]
    mix = [jnp.where(pr[l] >= _MINP, mix[l], _NEG) for l in range(4)]

    eosm = [(fi[l] == _EOS) & sv[l] for l in range(4)]
    ep = jnp.float32(0.0)
    ne = jnp.int32(0)
    for l in range(4):
      ep = ep + jnp.sum(jnp.where(eosm[l], pr[l], jnp.float32(0.0)))
      ne = ne + jnp.max(plsc.all_reduce_population_count(eosm[l]))
    ep_v = jnp.broadcast_to(ep, (_L,))
    eth = jnp.maximum(ep_v / jnp.full((_L,), _EOSR, jnp.float32),
                      jnp.float32(0.005))
    no_eos = ne == 0
    mix = [jnp.where((pr[l] >= eth) | no_eos, mix[l], _NEG)
           for l in range(4)]

    mxm = jnp.max(jnp.maximum(jnp.maximum(mix[0], mix[1]),
                              jnp.maximum(mix[2], mix[3])))
    cc = jnp.int32(0)
    chosen = jnp.int32(-1)
    for l in range(4):
      eq = mix[l] == mxm
      cl = jnp.cumsum(jnp.where(eq, 1, 0)) + cc
      first = eq & (cl == 1)
      chosen = jnp.maximum(chosen,
                           jnp.max(jnp.where(first, fi[l], jnp.int32(-1))))
      cc = cc + jnp.max(plsc.all_reduce_population_count(eq))

    greedy = ii[0][0]
    chosen = jnp.where(tmp < _EPS, greedy, chosen)
    store1(outb_v, r, chosen)
    return 0

  lax.fori_loop(0, _RW, row_body, 0)
  pltpu.sync_copy(outb_v, out_hbm.at[wid])


@jax.jit
def _run(logits, temperature, token_lengths, top_k, top_p):
  mesh = plsc.VectorSubcoreMesh(core_axis_name="c", subcore_axis_name="s")
  f = functools.partial(
      pl.kernel,
      out_type=jax.ShapeDtypeStruct((_NW, 8), jnp.int32),
      mesh=mesh,
      compiler_params=pltpu.CompilerParams(needs_layout_passes=False),
      scratch_types=[
          pltpu.VMEM((_V,), jnp.float32),
          pltpu.VMEM((_CAP,), jnp.float32),
          pltpu.VMEM((_CAP,), jnp.int32),
          pltpu.VMEM((_KP,), jnp.float32),
          pltpu.VMEM((_KP,), jnp.int32),
          pltpu.VMEM((_B,), jnp.float32),
          pltpu.VMEM((_B,), jnp.int32),
          pltpu.VMEM((_B,), jnp.float32),
          pltpu.VMEM((128,), jnp.int32),
          pltpu.VMEM((128,), jnp.int32),
          pltpu.VMEM((_KP,), jnp.int32),
          pltpu.VMEM((_KP,), jnp.int32),
          pltpu.VMEM((8,), jnp.int32),
          pltpu.SemaphoreType.DMA,
      ],
  )(_sc_body)
  return f(logits, temperature, token_lengths, top_k, top_p)


def kernel(logits, temperature, token_lengths, top_k, top_p):
  out = _run(logits.astype(jnp.float32),
             temperature.astype(jnp.float32),
             token_lengths.astype(jnp.int32),
             top_k.astype(jnp.int32),
             top_p.astype(jnp.float32))
  return out[:, :_RW].reshape(-1)


# speedup vs baseline: 41.5679x; 1.0070x over previous
import functools

import jax
import jax.numpy as jnp
from jax import lax
from jax.experimental import pallas as pl
from jax.experimental.pallas import tpu as pltpu
from jax.experimental.pallas import tpu_sc as plsc

_B = 128
_V = 100000
_K = 50
_KP = 64
_L = 16
_NVR = _V // _L
_CAP = 2048
_NW = 32
_RW = _B // _NW
_NEG = -3.0e38
_BIG = 3.0e38
_FIN = -1.0e37
_EPS = 1e-5
_EOS = 2
_MINP = 0.001
_EOSR = 100.0
_LNEG = -1.0e30


def _sc_body(logits_hbm, temp_hbm, tl_hbm, topk_hbm, topp_hbm, out_hbm,
             row_v, cand_i, top_v, top_i, temp_v, topk_v, topp_v,
             mark_v, pad_v, fidx_v, lens_v, outb_v, sem):
  cid = lax.axis_index("c")
  sid = lax.axis_index("s")
  wid = sid * 2 + cid

  pltpu.sync_copy(temp_hbm, temp_v)
  pltpu.sync_copy(topk_hbm, topk_v)
  pltpu.sync_copy(topp_hbm, topp_v)

  lanes = lax.iota(jnp.int32, _L)
  zeros_i = lanes * 0
  ones_i = zeros_i + 1
  neg16 = jnp.full((_L,), _NEG, jnp.float32)
  lane0 = lanes == 0

  def store1(ref, pos, val):
    plsc.store_scatter(ref, [jnp.broadcast_to(pos, (_L,))],
                       jnp.broadcast_to(val, (_L,)), mask=lane0)

  def load1(ref, pos):
    return plsc.load_gather(ref, [jnp.broadcast_to(pos, (_L,))])[0]

  def row_body(r, _carry):
    b = wid * _RW + r
    pltpu.sync_copy(logits_hbm.at[b], row_v)

    def p1(j, c):
      r0, r1, r2, r3 = c
      x = row_v[pl.ds(pl.multiple_of(j * _L, _L), _L)]
      m0 = jnp.maximum(r0, x)
      y = jnp.minimum(r0, x)
      m1 = jnp.maximum(r1, y)
      y = jnp.minimum(r1, y)
      m2 = jnp.maximum(r2, y)
      y = jnp.minimum(r2, y)
      m3 = jnp.maximum(r3, y)
      return (m0, m1, m2, m3)

    c0, c1, c2, c3 = lax.fori_loop(0, _NVR, p1, (neg16, neg16, neg16, neg16),
                                   unroll=8)

    def pt(_i, c):
      a0, a1, a2, a3 = c
      s = jnp.max(jnp.maximum(jnp.maximum(a0, a1), jnp.maximum(a2, a3)))
      a0 = jnp.where(a0 == s, _NEG, a0)
      a1 = jnp.where(a1 == s, _NEG, a1)
      a2 = jnp.where(a2 == s, _NEG, a2)
      a3 = jnp.where(a3 == s, _NEG, a3)
      return (a0, a1, a2, a3)

    a0, a1, a2, a3 = lax.fori_loop(0, _K - 1, pt, (c0, c1, c2, c3))
    t = jnp.max(jnp.maximum(jnp.maximum(a0, a1), jnp.maximum(a2, a3)))

    def p2(j, off):
      base = pl.multiple_of(j * _L, _L)
      x = row_v[pl.ds(base, _L)]
      msk = x >= t
      plsc.store_compressed(cand_i.at[pl.ds(off, _L)], lanes + j * _L,
                            mask=msk)
      cnt = jnp.max(plsc.all_reduce_population_count(msk))
      return jnp.minimum(off + cnt, _CAP - _L)

    off = lax.fori_loop(0, _NVR, p2, jnp.int32(0), unroll=8)
    nv = (off + _L - 1) >> 4

    for l in range(4):
      top_v[pl.ds(l * _L, _L)] = neg16
      top_i[pl.ds(l * _L, _L)] = zeros_i

    def cand_vals(j):
      ci = cand_i[pl.ds(pl.multiple_of(j * _L, _L), _L)]
      valid = (j * _L + lanes) < off
      v = plsc.load_gather(row_v, [ci], mask=valid)
      return jnp.where(valid, v, _NEG), ci

    def ext(q, _c):
      def fa(j, cur):
        v, _ = cand_vals(j)
        return jnp.maximum(cur, jnp.max(v))

      s = lax.fori_loop(0, nv, fa, jnp.float32(_NEG))

      def fb(j, pos):
        v, _ = cand_vals(j)
        f = jnp.max(plsc.all_reduce_ffs(v == s))
        return jnp.where((pos < 0) & (f < _L), j * _L + f, pos)

      pos = lax.fori_loop(0, nv, fb, jnp.int32(-1))
      gidx = load1(cand_i, pos)
      store1(top_v, q, s)
      store1(top_i, q, gidx)
      store1(row_v, gidx, jnp.float32(_NEG))
      return 0

    lax.fori_loop(0, _K, ext, 0)

    tmp = load1(temp_v, b)
    kk = jnp.clip(load1(topk_v, b), 1, _K)
    q1p = jnp.float32(1.0) - load1(topp_v, b)

    v = [top_v[pl.ds(l * _L, _L)] for l in range(4)]
    ii = [top_i[pl.ds(l * _L, _L)] for l in range(4)]
    rank = [lanes + l * _L for l in range(4)]

    tmp_v = jnp.broadcast_to(tmp, (_L,))
    w = [jnp.where(vl > _FIN, vl / tmp_v, _NEG) for vl in v]

    mm = [jnp.where(rank[l] < kk, w[l], _BIG) for l in range(4)]
    thr = jnp.min(jnp.minimum(jnp.minimum(mm[0], mm[1]),
                              jnp.minimum(mm[2], mm[3])))
    wk = [jnp.where(w[l] >= thr, w[l], _NEG) for l in range(4)]

    mx = jnp.max(jnp.maximum(jnp.maximum(wk[0], wk[1]),
                             jnp.maximum(wk[2], wk[3])))
    e = [jnp.exp(wk[l] - mx) for l in range(4)]
    tot = jnp.broadcast_to(jnp.sum(((e[0] + e[1]) + (e[2] + e[3]))), (_L,))
    d = [e[l] / tot for l in range(4)]

    asc = [jnp.flip(d[3 - l], axis=0) for l in range(4)]
    base = jnp.float32(0.0)
    acum = []
    for l in range(4):
      acum.append(jnp.cumsum(asc[l]) + base)
      base = base + jnp.sum(asc[l])
    s_desc = [jnp.flip(acum[3 - l], axis=0) for l in range(4)]

    cond = [s_desc[l] <= q1p for l in range(4)]
    cond[0] = cond[0] & (lanes > 0)
    w2 = [jnp.where(cond[l], _NEG, wk[l]) for l in range(4)]

    mx2 = jnp.max(jnp.maximum(jnp.maximum(w2[0], w2[1]),
                              jnp.maximum(w2[2], w2[3])))
    e2 = [jnp.exp(w2[l] - mx2) for l in range(4)]
    tot2 = jnp.broadcast_to(jnp.sum(((e2[0] + e2[1]) + (e2[2] + e2[3]))),
                            (_L,))
    pr = [e2[l] / tot2 for l in range(4)]

    sv = [w2[l] > _FIN for l in range(4)]
    nsur = jnp.int32(0)
    for l in range(4):
      nsur = nsur + jnp.max(plsc.all_reduce_population_count(sv[l]))
    mpad = _K - nsur

    for l2 in range(8):
      mark_v[pl.ds(l2 * _L, _L)] = zeros_i
    for l in range(4):
      plsc.store_scatter(mark_v, [ii[l]], ones_i,
                         mask=sv[l] & (ii[l] < 128))

    def pp_body(l2, fc):
      mk = mark_v[pl.ds(pl.multiple_of(l2 * _L, _L), _L)]
      free = mk == 0
      fcum = jnp.cumsum(jnp.where(free, 1, 0)) + fc
      sel = free & (fcum <= mpad)
      woff = jnp.minimum(fc, mpad)
      plsc.store_compressed(pad_v.at[pl.ds(woff, _L)], lanes + l2 * _L,
                            mask=sel)
      return fc + jnp.max(plsc.all_reduce_population_count(free))

    lax.fori_loop(0, 8, pp_body, jnp.int32(0))

    hcarry = jnp.int32(0)
    fi = []
    for l in range(4):
      hole = (rank[l] < _K) & jnp.logical_not(sv[l])
      hc = jnp.cumsum(jnp.where(hole, 1, 0)) + hcarry
      g = plsc.load_gather(pad_v, [hc - 1], mask=hole)
      f = jnp.where(hole, g, ii[l])
      f = jnp.where(rank[l] < _K, f, 0)
      f = jnp.clip(f, 0, _V - 1)
      fi.append(f)
      fidx_v[pl.ds(l * _L, _L)] = f
      hcarry = hcarry + jnp.max(plsc.all_reduce_population_count(hole))

    pltpu.async_copy(tl_hbm.at[fidx_v], lens_v, sem).wait()

    lw = [jnp.where(rank[l] < _K,
                    lens_v[pl.ds(l * _L, _L)].astype(jnp.float32), _LNEG)
          for l in range(4)]
    lmx = jnp.max(jnp.maximum(jnp.maximum(lw[0], lw[1]),
                              jnp.maximum(lw[2], lw[3])))
    le = [jnp.exp(lw[l] - lmx) for l in range(4)]
    ltot = jnp.broadcast_to(jnp.sum(((le[0] + le[1]) + (le[2] + le[3]))),
                            (_L,))
    ls = [le[l] / ltot for l in range(4)]

    mix = [jnp.float32(0.5) * pr[l] + jnp.float32(0.5) * ls[l]
           for l in range(4)]
    mix = [jnp.where(pr[l] >= _MINP, mix[l], _NEG) for l in range(4)]

    eosm = [(fi[l] == _EOS) & sv[l] for l in range(4)]
    ep = jnp.float32(0.0)
    ne = jnp.int32(0)
    for l in range(4):
      ep = ep + jnp.sum(jnp.where(eosm[l], pr[l], jnp.float32(0.0)))
      ne = ne + jnp.max(plsc.all_reduce_population_count(eosm[l]))
    ep_v = jnp.broadcast_to(ep, (_L,))
    eth = jnp.maximum(ep_v / jnp.full((_L,), _EOSR, jnp.float32),
                      jnp.float32(0.005))
    no_eos = ne == 0
    mix = [jnp.where((pr[l] >= eth) | no_eos, mix[l], _NEG)
           for l in range(4)]

    mxm = jnp.max(jnp.maximum(jnp.maximum(mix[0], mix[1]),
                              jnp.maximum(mix[2], mix[3])))
    cc = jnp.int32(0)
    chosen = jnp.int32(-1)
    for l in range(4):
      eq = mix[l] == mxm
      cl = jnp.cumsum(jnp.where(eq, 1, 0)) + cc
      first = eq & (cl == 1)
      chosen = jnp.maximum(chosen,
                           jnp.max(jnp.where(first, fi[l], jnp.int32(-1))))
      cc = cc + jnp.max(plsc.all_reduce_population_count(eq))

    greedy = ii[0][0]
    chosen = jnp.where(tmp < _EPS, greedy, chosen)
    store1(outb_v, r, chosen)
    return 0

  lax.fori_loop(0, _RW, row_body, 0)
  pltpu.sync_copy(outb_v, out_hbm.at[wid])


@jax.jit
def _run(logits, temperature, token_lengths, top_k, top_p):
  mesh = plsc.VectorSubcoreMesh(core_axis_name="c", subcore_axis_name="s")
  f = functools.partial(
      pl.kernel,
      out_type=jax.ShapeDtypeStruct((_NW, 8), jnp.int32),
      mesh=mesh,
      compiler_params=pltpu.CompilerParams(needs_layout_passes=False),
      scratch_types=[
          pltpu.VMEM((_V,), jnp.float32),
          pltpu.VMEM((_CAP,), jnp.int32),
          pltpu.VMEM((_KP,), jnp.float32),
          pltpu.VMEM((_KP,), jnp.int32),
          pltpu.VMEM((_B,), jnp.float32),
          pltpu.VMEM((_B,), jnp.int32),
          pltpu.VMEM((_B,), jnp.float32),
          pltpu.VMEM((128,), jnp.int32),
          pltpu.VMEM((128,), jnp.int32),
          pltpu.VMEM((_KP,), jnp.int32),
          pltpu.VMEM((_KP,), jnp.int32),
          pltpu.VMEM((8,), jnp.int32),
          pltpu.SemaphoreType.DMA,
      ],
  )(_sc_body)
  return f(logits, temperature, token_lengths, top_k, top_p)


def kernel(logits, temperature, token_lengths, top_k, top_p):
  out = _run(logits.astype(jnp.float32),
             temperature.astype(jnp.float32),
             token_lengths.astype(jnp.int32),
             top_k.astype(jnp.int32),
             top_p.astype(jnp.float32))
  return out[:, :_RW].reshape(-1)


# speedup vs baseline: 48.8344x; 1.1748x over previous
import functools

import jax
import jax.numpy as jnp
from jax import lax
from jax.experimental import pallas as pl
from jax.experimental.pallas import tpu as pltpu
from jax.experimental.pallas import tpu_sc as plsc

_B = 128
_V = 100000
_K = 50
_KP = 64
_L = 16
_NVR = _V // _L
_CAP = 2048
_NW = 32
_RW = _B // _NW
_NEG = -3.0e38
_BIG = 3.0e38
_FIN = -1.0e37
_EPS = 1e-5
_EOS = 2
_MINP = 0.001
_EOSR = 100.0
_LNEG = -1.0e30


def _sc_body(logits_hbm, temp_hbm, tl_hbm, topk_hbm, topp_hbm, out_hbm,
             row_v, cand_i, top_v, top_i, temp_v, topk_v, topp_v,
             mark_v, pad_v, fidx_v, lens_v, outb_v, sem):
  cid = lax.axis_index("c")
  sid = lax.axis_index("s")
  wid = sid * 2 + cid

  pltpu.sync_copy(temp_hbm, temp_v)
  pltpu.sync_copy(topk_hbm, topk_v)
  pltpu.sync_copy(topp_hbm, topp_v)

  lanes = lax.iota(jnp.int32, _L)
  zeros_i = lanes * 0
  ones_i = zeros_i + 1
  neg16 = jnp.full((_L,), _NEG, jnp.float32)
  lane0 = lanes == 0

  def store1(ref, pos, val):
    plsc.store_scatter(ref, [jnp.broadcast_to(pos, (_L,))],
                       jnp.broadcast_to(val, (_L,)), mask=lane0)

  def load1(ref, pos):
    return plsc.load_gather(ref, [jnp.broadcast_to(pos, (_L,))])[0]

  def row_body(r, _carry):
    b = wid * _RW + r
    pltpu.sync_copy(logits_hbm.at[b], row_v)

    def ins4(c, x):
      r0, r1, r2, r3 = c
      m0 = jnp.maximum(r0, x)
      y = jnp.minimum(r0, x)
      m1 = jnp.maximum(r1, y)
      y = jnp.minimum(r1, y)
      m2 = jnp.maximum(r2, y)
      y = jnp.minimum(r2, y)
      m3 = jnp.maximum(r3, y)
      return (m0, m1, m2, m3)

    def p1(j, cs):
      out = []
      for s in range(4):
        x = row_v[pl.ds(pl.multiple_of((j * 4 + s) * _L, _L), _L)]
        out.append(ins4(cs[s], x))
      return tuple(out)

    init4 = (neg16, neg16, neg16, neg16)
    cs = lax.fori_loop(0, _NVR // 4, p1, (init4, init4, init4, init4),
                       unroll=2)
    acc = cs[0]
    for s in range(_NVR % 4):
      x = row_v[pl.ds((_NVR // 4 * 4 + s) * _L, _L)]
      acc = ins4(acc, x)
    for s in range(1, 4):
      for r4 in range(4):
        acc = ins4(acc, cs[s][r4])
    c0, c1, c2, c3 = acc

    def pt(_i, c):
      a0, a1, a2, a3 = c
      s = jnp.max(jnp.maximum(jnp.maximum(a0, a1), jnp.maximum(a2, a3)))
      a0 = jnp.where(a0 == s, _NEG, a0)
      a1 = jnp.where(a1 == s, _NEG, a1)
      a2 = jnp.where(a2 == s, _NEG, a2)
      a3 = jnp.where(a3 == s, _NEG, a3)
      return (a0, a1, a2, a3)

    a0, a1, a2, a3 = lax.fori_loop(0, _K - 1, pt, (c0, c1, c2, c3))
    t = jnp.max(jnp.maximum(jnp.maximum(a0, a1), jnp.maximum(a2, a3)))

    def p2(j, off):
      base = pl.multiple_of(j * _L, _L)
      x = row_v[pl.ds(base, _L)]
      msk = x >= t
      plsc.store_compressed(cand_i.at[pl.ds(off, _L)], lanes + j * _L,
                            mask=msk)
      cnt = plsc.all_reduce_population_count(msk)[0]
      return jnp.minimum(off + cnt, _CAP - _L)

    off = lax.fori_loop(0, _NVR, p2, jnp.int32(0), unroll=8)
    nv = (off + _L - 1) >> 4

    for l in range(4):
      top_v[pl.ds(l * _L, _L)] = neg16
      top_i[pl.ds(l * _L, _L)] = zeros_i

    def cand_vals(j):
      ci = cand_i[pl.ds(pl.multiple_of(j * _L, _L), _L)]
      valid = (j * _L + lanes) < off
      v = plsc.load_gather(row_v, [ci], mask=valid)
      return jnp.where(valid, v, _NEG), ci

    def ext(q, _c):
      def fa(j, cur):
        v, _ = cand_vals(j)
        return jnp.maximum(cur, v)

      s = jnp.max(lax.fori_loop(0, nv, fa, neg16))

      def fb(j, pos):
        v, _ = cand_vals(j)
        f = plsc.all_reduce_ffs(v == s)[0]
        return jnp.where((pos < 0) & (f < _L), j * _L + f, pos)

      pos = lax.fori_loop(0, nv, fb, jnp.int32(-1))
      gidx = load1(cand_i, pos)
      store1(top_v, q, s)
      store1(top_i, q, gidx)
      store1(row_v, gidx, jnp.float32(_NEG))
      return 0

    lax.fori_loop(0, _K, ext, 0)

    tmp = load1(temp_v, b)
    kk = jnp.clip(load1(topk_v, b), 1, _K)
    q1p = jnp.float32(1.0) - load1(topp_v, b)

    v = [top_v[pl.ds(l * _L, _L)] for l in range(4)]
    ii = [top_i[pl.ds(l * _L, _L)] for l in range(4)]
    rank = [lanes + l * _L for l in range(4)]

    tmp_v = jnp.broadcast_to(tmp, (_L,))
    w = [jnp.where(vl > _FIN, vl / tmp_v, _NEG) for vl in v]

    mm = [jnp.where(rank[l] < kk, w[l], _BIG) for l in range(4)]
    thr = jnp.min(jnp.minimum(jnp.minimum(mm[0], mm[1]),
                              jnp.minimum(mm[2], mm[3])))
    wk = [jnp.where(w[l] >= thr, w[l], _NEG) for l in range(4)]

    mx = jnp.max(jnp.maximum(jnp.maximum(wk[0], wk[1]),
                             jnp.maximum(wk[2], wk[3])))
    e = [jnp.exp(wk[l] - mx) for l in range(4)]
    tot = jnp.broadcast_to(jnp.sum(((e[0] + e[1]) + (e[2] + e[3]))), (_L,))
    d = [e[l] / tot for l in range(4)]

    asc = [jnp.flip(d[3 - l], axis=0) for l in range(4)]
    base = jnp.float32(0.0)
    acum = []
    for l in range(4):
      acum.append(jnp.cumsum(asc[l]) + base)
      base = base + jnp.sum(asc[l])
    s_desc = [jnp.flip(acum[3 - l], axis=0) for l in range(4)]

    cond = [s_desc[l] <= q1p for l in range(4)]
    cond[0] = cond[0] & (lanes > 0)
    w2 = [jnp.where(cond[l], _NEG, wk[l]) for l in range(4)]

    mx2 = jnp.max(jnp.maximum(jnp.maximum(w2[0], w2[1]),
                              jnp.maximum(w2[2], w2[3])))
    e2 = [jnp.exp(w2[l] - mx2) for l in range(4)]
    tot2 = jnp.broadcast_to(jnp.sum(((e2[0] + e2[1]) + (e2[2] + e2[3]))),
                            (_L,))
    pr = [e2[l] / tot2 for l in range(4)]

    sv = [w2[l] > _FIN for l in range(4)]
    nsur = jnp.int32(0)
    for l in range(4):
      nsur = nsur + jnp.max(plsc.all_reduce_population_count(sv[l]))
    mpad = _K - nsur

    for l2 in range(8):
      mark_v[pl.ds(l2 * _L, _L)] = zeros_i
    for l in range(4):
      plsc.store_scatter(mark_v, [ii[l]], ones_i,
                         mask=sv[l] & (ii[l] < 128))

    def pp_body(l2, fc):
      mk = mark_v[pl.ds(pl.multiple_of(l2 * _L, _L), _L)]
      free = mk == 0
      fcum = jnp.cumsum(jnp.where(free, 1, 0)) + fc
      sel = free & (fcum <= mpad)
      woff = jnp.minimum(fc, mpad)
      plsc.store_compressed(pad_v.at[pl.ds(woff, _L)], lanes + l2 * _L,
                            mask=sel)
      return fc + jnp.max(plsc.all_reduce_population_count(free))

    lax.fori_loop(0, 8, pp_body, jnp.int32(0))

    hcarry = jnp.int32(0)
    fi = []
    for l in range(4):
      hole = (rank[l] < _K) & jnp.logical_not(sv[l])
      hc = jnp.cumsum(jnp.where(hole, 1, 0)) + hcarry
      g = plsc.load_gather(pad_v, [hc - 1], mask=hole)
      f = jnp.where(hole, g, ii[l])
      f = jnp.where(rank[l] < _K, f, 0)
      f = jnp.clip(f, 0, _V - 1)
      fi.append(f)
      fidx_v[pl.ds(l * _L, _L)] = f
      hcarry = hcarry + jnp.max(plsc.all_reduce_population_count(hole))

    pltpu.async_copy(tl_hbm.at[fidx_v], lens_v, sem).wait()

    lw = [jnp.where(rank[l] < _K,
                    lens_v[pl.ds(l * _L, _L)].astype(jnp.float32), _LNEG)
          for l in range(4)]
    lmx = jnp.max(jnp.maximum(jnp.maximum(lw[0], lw[1]),
                              jnp.maximum(lw[2], lw[3])))
    le = [jnp.exp(lw[l] - lmx) for l in range(4)]
    ltot = jnp.broadcast_to(jnp.sum(((le[0] + le[1]) + (le[2] + le[3]))),
                            (_L,))
    ls = [le[l] / ltot for l in range(4)]

    mix = [jnp.float32(0.5) * pr[l] + jnp.float32(0.5) * ls[l]
           for l in range(4)]
    mix = [jnp.where(pr[l] >= _MINP, mix[l], _NEG) for l in range(4)]

    eosm = [(fi[l] == _EOS) & sv[l] for l in range(4)]
    ep = jnp.float32(0.0)
    ne = jnp.int32(0)
    for l in range(4):
      ep = ep + jnp.sum(jnp.where(eosm[l], pr[l], jnp.float32(0.0)))
      ne = ne + jnp.max(plsc.all_reduce_population_count(eosm[l]))
    ep_v = jnp.broadcast_to(ep, (_L,))
    eth = jnp.maximum(ep_v / jnp.full((_L,), _EOSR, jnp.float32),
                      jnp.float32(0.005))
    no_eos = ne == 0
    mix = [jnp.where((pr[l] >= eth) | no_eos, mix[l], _NEG)
           for l in range(4)]

    mxm = jnp.max(jnp.maximum(jnp.maximum(mix[0], mix[1]),
                              jnp.maximum(mix[2], mix[3])))
    cc = jnp.int32(0)
    chosen = jnp.int32(-1)
    for l in range(4):
      eq = mix[l] == mxm
      cl = jnp.cumsum(jnp.where(eq, 1, 0)) + cc
      first = eq & (cl == 1)
      chosen = jnp.maximum(chosen,
                           jnp.max(jnp.where(first, fi[l], jnp.int32(-1))))
      cc = cc + jnp.max(plsc.all_reduce_population_count(eq))

    greedy = ii[0][0]
    chosen = jnp.where(tmp < _EPS, greedy, chosen)
    store1(outb_v, r, chosen)
    return 0

  lax.fori_loop(0, _RW, row_body, 0)
  pltpu.sync_copy(outb_v, out_hbm.at[wid])


@jax.jit
def _run(logits, temperature, token_lengths, top_k, top_p):
  mesh = plsc.VectorSubcoreMesh(core_axis_name="c", subcore_axis_name="s")
  f = functools.partial(
      pl.kernel,
      out_type=jax.ShapeDtypeStruct((_NW, 8), jnp.int32),
      mesh=mesh,
      compiler_params=pltpu.CompilerParams(needs_layout_passes=False),
      scratch_types=[
          pltpu.VMEM((_V,), jnp.float32),
          pltpu.VMEM((_CAP,), jnp.int32),
          pltpu.VMEM((_KP,), jnp.float32),
          pltpu.VMEM((_KP,), jnp.int32),
          pltpu.VMEM((_B,), jnp.float32),
          pltpu.VMEM((_B,), jnp.int32),
          pltpu.VMEM((_B,), jnp.float32),
          pltpu.VMEM((128,), jnp.int32),
          pltpu.VMEM((128,), jnp.int32),
          pltpu.VMEM((_KP,), jnp.int32),
          pltpu.VMEM((_KP,), jnp.int32),
          pltpu.VMEM((8,), jnp.int32),
          pltpu.SemaphoreType.DMA,
      ],
  )(_sc_body)
  return f(logits, temperature, token_lengths, top_k, top_p)


def kernel(logits, temperature, token_lengths, top_k, top_p):
  out = _run(logits.astype(jnp.float32),
             temperature.astype(jnp.float32),
             token_lengths.astype(jnp.int32),
             top_k.astype(jnp.int32),
             top_p.astype(jnp.float32))
  return out[:, :_RW].reshape(-1)
